# baseline (speedup 1.0000x reference)
import functools

import jax
import jax.numpy as jnp
from jax import lax
from jax.experimental import pallas as pl
from jax.experimental.pallas import tpu as pltpu
from jax.experimental.pallas import tpu_sc as plsc

_K = 16
_GRID_LEN = 32
_QB = 256


def _tc_body(x_ref, out_ref):
    qb = pl.program_id(1)
    xb = x_ref[0]
    P = xb.shape[1]
    x01 = xb[0:2, :]

    mn0 = jnp.min(xb[0:1, :])
    mx0 = jnp.max(xb[0:1, :])
    mn1 = jnp.min(xb[1:2, :])
    mx1 = jnp.max(xb[1:2, :])
    j = lax.broadcasted_iota(jnp.int32, (_QB, 2), 0) + qb * _QB
    c = lax.broadcasted_iota(jnp.int32, (_QB, 2), 1)
    mesh = jnp.where(c == 0, j % _GRID_LEN, j // _GRID_LEN).astype(jnp.float32)
    mesh = mesh / jnp.float32(_GRID_LEN)
    mn = jnp.where(c == 0, mn0, mn1)
    mx = jnp.where(c == 0, mx0, mx1)
    q = mesh * (mx - mn) + mn

    inner = -2.0 * lax.dot_general(
        q, x01, (((1,), (0,)), ((), ())), preferred_element_type=jnp.float32
    )
    xsq01 = jnp.sum(x01 * x01, axis=0, keepdims=True)
    qsq = jnp.sum(q * q, axis=1, keepdims=True)
    d1 = (-xsq01) - inner - qsq
    idx1 = jnp.argmax(d1, axis=-1).astype(jnp.int32)

    lid = lax.broadcasted_iota(jnp.int32, (_QB, P), 1)
    ohm = lid == idx1[:, None]
    st = jnp.concatenate(
        [
            jnp.sum(
                jnp.where(ohm, xb[cc : cc + 1, :], 0.0), axis=-1, keepdims=True
            )
            for cc in range(3)
        ],
        axis=1,
    )

    inner2 = -2.0 * lax.dot_general(
        st, xb, (((1,), (0,)), ((), ())), preferred_element_type=jnp.float32
    )
    xsq = jnp.sum(xb * xb, axis=0, keepdims=True)
    ssq = jnp.sum(st * st, axis=1, keepdims=True)
    d = (-xsq) - inner2 - ssq

    neg_inf = jnp.float32(float("-inf"))
    for k in range(_K):
        idx = jnp.argmax(d, axis=-1).astype(jnp.int32)
        out_ref[0, :, k] = idx
        d = jnp.where(lid == idx[:, None], neg_inf, d)


def _tc_knn(x):
    B, C, P = x.shape
    S = _GRID_LEN * _GRID_LEN
    return pl.pallas_call(
        _tc_body,
        grid=(B, S // _QB),
        in_specs=[pl.BlockSpec((1, C, P), lambda b, qb: (b, 0, 0))],
        out_specs=pl.BlockSpec((1, _QB, _K), lambda b, qb: (b, qb, 0)),
        out_shape=jax.ShapeDtypeStruct((B, S, _K), jnp.int32),
    )(x)


_IDXB = 128


def _sc_gather_call(table, gidx):
    M = gidx.shape[0]
    info = plsc.get_sparse_core_info()
    nw = info.num_cores * info.num_subcores
    chunk = M // nw
    assert chunk % _IDXB == 0 and M % nw == 0
    nb = chunk // _IDXB
    mesh = plsc.VectorSubcoreMesh(core_axis_name="c", subcore_axis_name="s")

    @functools.partial(
        pl.kernel,
        mesh=mesh,
        out_type=jax.ShapeDtypeStruct((M, 16), jnp.float32),
        compiler_params=pltpu.CompilerParams(use_tc_tiling_on_sc=False),
        scratch_types=[
            pltpu.VMEM((nb, _IDXB), jnp.int32),
            pltpu.VMEM((chunk, 16), jnp.float32),
            pltpu.SemaphoreType.DMA,
        ],
    )
    def k(table_hbm, idx_hbm, out_hbm, idx_v, rows_v, sem):
        wid = lax.axis_index("c") * info.num_subcores + lax.axis_index("s")
        pltpu.sync_copy(idx_hbm.at[pl.ds(wid * nb, nb)], idx_v)
        copies = []
        for r in range(nb):
            copies.append(
                pltpu.async_copy(
                    table_hbm.at[idx_v.at[r]],
                    rows_v.at[pl.ds(r * _IDXB, _IDXB)],
                    sem,
                )
            )
        for cp in copies:
            cp.wait()
        pltpu.sync_copy(rows_v, out_hbm.at[pl.ds(wid * chunk, chunk)])

    return k(table, gidx.reshape(M // _IDXB, _IDXB))


def kernel(x, s_num):
    B, C, P = x.shape
    S = _GRID_LEN * _GRID_LEN
    table = jnp.pad(jnp.swapaxes(x, 1, 2), ((0, 0), (0, 0), (0, 16 - C)))
    table = table.reshape(B * P, 16)
    boff = (jnp.arange(B, dtype=jnp.int32) * P)[:, None]

    idx2 = _tc_knn(x)
    feat = _sc_gather_call(table, (idx2.reshape(B, S * _K) + boff).reshape(-1))
    return jnp.transpose(feat[:, :C].reshape(B, S, _K, C), (0, 3, 1, 2))


# speedup vs baseline: 1.2642x; 1.2642x over previous
import functools

import jax
import jax.numpy as jnp
from jax import lax
from jax.experimental import pallas as pl
from jax.experimental.pallas import tpu as pltpu
from jax.experimental.pallas import tpu_sc as plsc

_K = 16
_GRID_LEN = 32
_QB = 256


def _stage_a_body(x_ref, out_ref):
    qb = pl.program_id(1)
    xb = x_ref[0]
    x01 = xb[0:2, :]

    mn0 = jnp.min(xb[0:1, :])
    mx0 = jnp.max(xb[0:1, :])
    mn1 = jnp.min(xb[1:2, :])
    mx1 = jnp.max(xb[1:2, :])
    j = lax.broadcasted_iota(jnp.int32, (_QB, 2), 0) + qb * _QB
    c = lax.broadcasted_iota(jnp.int32, (_QB, 2), 1)
    mesh = jnp.where(c == 0, j % _GRID_LEN, j // _GRID_LEN).astype(jnp.float32)
    mesh = mesh / jnp.float32(_GRID_LEN)
    mn = jnp.where(c == 0, mn0, mn1)
    mx = jnp.where(c == 0, mx0, mx1)
    q = mesh * (mx - mn) + mn

    inner = -2.0 * lax.dot_general(
        q, x01, (((1,), (0,)), ((), ())), preferred_element_type=jnp.float32
    )
    xsq01 = jnp.sum(x01 * x01, axis=0, keepdims=True)
    qsq = jnp.sum(q * q, axis=1, keepdims=True)
    d1 = (-xsq01) - inner - qsq
    idx1 = jnp.argmax(d1, axis=-1).astype(jnp.int32)
    out_ref[0] = idx1[:, None]


def _stage_a(x):
    B, C, P = x.shape
    S = _GRID_LEN * _GRID_LEN
    return pl.pallas_call(
        _stage_a_body,
        grid=(B, S // _QB),
        in_specs=[pl.BlockSpec((1, C, P), lambda b, qb: (b, 0, 0))],
        out_specs=pl.BlockSpec((1, _QB, 1), lambda b, qb: (b, qb, 0)),
        out_shape=jax.ShapeDtypeStruct((B, S, 1), jnp.int32),
    )(x)


def _stage_c_body(x_ref, s_ref, out_ref):
    xb = x_ref[0]
    P = xb.shape[1]
    st = s_ref[0]

    inner2 = -2.0 * lax.dot_general(
        st, xb, (((1,), (0,)), ((), ())), preferred_element_type=jnp.float32
    )
    xsq = jnp.sum(xb * xb, axis=0, keepdims=True)
    ssq = jnp.sum(st * st, axis=1, keepdims=True)
    d = (-xsq) - inner2 - ssq

    lid = lax.broadcasted_iota(jnp.int32, (_QB, P), 1)
    neg_inf = jnp.float32(float("-inf"))
    for k in range(_K):
        idx = jnp.argmax(d, axis=-1).astype(jnp.int32)
        out_ref[0, :, k] = idx
        d = jnp.where(lid == idx[:, None], neg_inf, d)


def _tc_knn(x, sel_t):
    B, C, P = x.shape
    S = sel_t.shape[1]
    return pl.pallas_call(
        _stage_c_body,
        grid=(B, S // _QB),
        in_specs=[
            pl.BlockSpec((1, C, P), lambda b, qb: (b, 0, 0)),
            pl.BlockSpec((1, _QB, C), lambda b, qb: (b, qb, 0)),
        ],
        out_specs=pl.BlockSpec((1, _QB, _K), lambda b, qb: (b, qb, 0)),
        out_shape=jax.ShapeDtypeStruct((B, S, _K), jnp.int32),
    )(x, sel_t)


_IDXB = 128


def _sc_gather_call(table, gidx):
    M = gidx.shape[0]
    info = plsc.get_sparse_core_info()
    nw = info.num_cores * info.num_subcores
    chunk = M // nw
    assert chunk % _IDXB == 0 and M % nw == 0
    nb = chunk // _IDXB
    mesh = plsc.VectorSubcoreMesh(core_axis_name="c", subcore_axis_name="s")

    @functools.partial(
        pl.kernel,
        mesh=mesh,
        out_type=jax.ShapeDtypeStruct((M, 16), jnp.float32),
        compiler_params=pltpu.CompilerParams(use_tc_tiling_on_sc=False),
        scratch_types=[
            pltpu.VMEM((nb, _IDXB), jnp.int32),
            pltpu.VMEM((chunk, 16), jnp.float32),
            pltpu.SemaphoreType.DMA,
        ],
    )
    def k(table_hbm, idx_hbm, out_hbm, idx_v, rows_v, sem):
        wid = lax.axis_index("c") * info.num_subcores + lax.axis_index("s")
        pltpu.sync_copy(idx_hbm.at[pl.ds(wid * nb, nb)], idx_v)
        copies = []
        for r in range(nb):
            copies.append(
                pltpu.async_copy(
                    table_hbm.at[idx_v.at[r]],
                    rows_v.at[pl.ds(r * _IDXB, _IDXB)],
                    sem,
                )
            )
        for cp in copies:
            cp.wait()
        pltpu.sync_copy(rows_v, out_hbm.at[pl.ds(wid * chunk, chunk)])

    return k(table, gidx.reshape(M // _IDXB, _IDXB))


def kernel(x, s_num):
    B, C, P = x.shape
    S = _GRID_LEN * _GRID_LEN
    table = jnp.pad(jnp.swapaxes(x, 1, 2), ((0, 0), (0, 0), (0, 16 - C)))
    table = table.reshape(B * P, 16)
    boff = (jnp.arange(B, dtype=jnp.int32) * P)[:, None]

    idx1 = _stage_a(x).reshape(B, S)
    sel_t = _sc_gather_call(table, (idx1 + boff).reshape(-1))
    sel_t = sel_t[:, :C].reshape(B, S, C)
    idx2 = _tc_knn(x, sel_t)
    feat = _sc_gather_call(table, (idx2.reshape(B, S * _K) + boff).reshape(-1))
    return jnp.transpose(feat[:, :C].reshape(B, S, _K, C), (0, 3, 1, 2))
